# Initial kernel scaffold; baseline (speedup 1.0000x reference)
import functools

import jax
import jax.numpy as jnp
from jax import lax
from jax.experimental import pallas as pl
from jax.experimental.pallas import tpu as pltpu
from jax.experimental.pallas import tpu_sc as plsc

N = 10000
E = 320000
NSTEP = 3
NFEAT = 128
NHID = 128
NCLASS = 64
HALF = NHID // 2

NC = 2
NS = 16
LANES = 16

E3 = NSTEP * E
EPT1 = E3 // NS
C1 = 96
NCH1 = EPT1 // C1
ROWS1 = NSTEP * N
RPT1 = ROWS1 // NS
ZROWS = 125

EPT2 = E // (NC * NS)
C2 = 80
NCH2 = EPT2 // C2
RPT2 = N // NS
ZROWS2 = 125

RB = 1000


def _support_body(x_ref, w_ref, o_ref):
    o_ref[...] = jnp.dot(x_ref[...], w_ref[0],
                         preferred_element_type=jnp.float32)


def _supports(x, W1):
    return pl.pallas_call(
        _support_body,
        grid=(NC, NSTEP, N // RB),
        in_specs=[
            pl.BlockSpec((RB, NFEAT), lambda c, s, r: (r, 0)),
            pl.BlockSpec((1, NFEAT, HALF), lambda c, s, r: (s, 0, c)),
        ],
        out_specs=pl.BlockSpec(
            (RB, HALF), lambda c, s, r: ((c * NSTEP + s) * (N // RB) + r, 0)),
        out_shape=jax.ShapeDtypeStruct((NC * ROWS1, HALF), jnp.float32),
    )(x, W1)


def _spmm1(sup, srcA, dstA):
    mesh = plsc.VectorSubcoreMesh(core_axis_name="c", subcore_axis_name="s")

    @functools.partial(
        pl.kernel,
        mesh=mesh,
        out_type=jax.ShapeDtypeStruct((NC * ROWS1, HALF), jnp.float32),
        scratch_types=[
            pltpu.VMEM((C1,), jnp.int32),
            pltpu.VMEM((C1,), jnp.int32),
            pltpu.VMEM((C1, HALF), jnp.float32),
            pltpu.VMEM((ZROWS, HALF), jnp.float32),
            pltpu.VMEM_SHARED((ROWS1, HALF), jnp.float32),
        ],
    )
    def k(sup_hbm, src_hbm, dst_hbm, out_hbm, src_v, dst_v, rows_v, zbuf, acc):
        c = lax.axis_index("c")
        s = lax.axis_index("s")

        @pl.loop(0, ZROWS)
        def _(i):
            for j in range(HALF // LANES):
                zbuf[i, pl.ds(j * LANES, LANES)] = jnp.zeros(
                    (LANES,), jnp.float32)

        for kk in range(RPT1 // ZROWS):
            pltpu.sync_copy(
                zbuf, acc.at[pl.ds(s * RPT1 + kk * ZROWS, ZROWS)])
        plsc.subcore_barrier()

        base_e = s * EPT1
        off_core = c * ROWS1

        @pl.loop(0, NCH1)
        def _(g):
            off = base_e + g * C1
            pltpu.sync_copy(src_hbm.at[pl.ds(off, C1)], src_v)
            pltpu.sync_copy(dst_hbm.at[pl.ds(off, C1)], dst_v)
            for j in range(C1 // LANES):
                sl = pl.ds(j * LANES, LANES)
                src_v[sl] = src_v[sl] + off_core
            pltpu.sync_copy(sup_hbm.at[src_v], rows_v)
            pltpu.sync_copy(rows_v, acc.at[dst_v], add=True)

        plsc.subcore_barrier()
        pltpu.sync_copy(acc.at[pl.ds(s * RPT1, RPT1)],
                        out_hbm.at[pl.ds(off_core + s * RPT1, RPT1)])

    return k(sup, srcA, dstA)


def _layer2_body(h_ref, b1_ref, w2_ref, o_ref):
    h = jnp.maximum(h_ref[...] + b1_ref[...], 0.0)
    o_ref[...] = jnp.dot(h, w2_ref[...], preferred_element_type=jnp.float32)


def _layer2(h_pre, b1f, W2):
    return pl.pallas_call(
        _layer2_body,
        grid=(N // RB,),
        in_specs=[
            pl.BlockSpec((RB, NSTEP * NHID), lambda r: (r, 0)),
            pl.BlockSpec((1, NSTEP * NHID), lambda r: (0, 0)),
            pl.BlockSpec((NSTEP * NHID, NCLASS), lambda r: (0, 0)),
        ],
        out_specs=pl.BlockSpec((RB, NCLASS), lambda r: (r, 0)),
        out_shape=jax.ShapeDtypeStruct((N, NCLASS), jnp.float32),
    )(h_pre, b1f, W2)


def _spmm2(sup2, src2, dst2):
    mesh = plsc.VectorSubcoreMesh(core_axis_name="c", subcore_axis_name="s")

    @functools.partial(
        pl.kernel,
        mesh=mesh,
        out_type=jax.ShapeDtypeStruct((NC * N, NCLASS), jnp.float32),
        scratch_types=[
            pltpu.VMEM((C2,), jnp.int32),
            pltpu.VMEM((C2,), jnp.int32),
            pltpu.VMEM((C2, NCLASS), jnp.float32),
            pltpu.VMEM((ZROWS2, NCLASS), jnp.float32),
            pltpu.VMEM_SHARED((N, NCLASS), jnp.float32),
        ],
    )
    def k(sup_hbm, src_hbm, dst_hbm, out_hbm, src_v, dst_v, rows_v, zbuf, acc):
        c = lax.axis_index("c")
        s = lax.axis_index("s")

        @pl.loop(0, ZROWS2)
        def _(i):
            for j in range(NCLASS // LANES):
                zbuf[i, pl.ds(j * LANES, LANES)] = jnp.zeros(
                    (LANES,), jnp.float32)

        for kk in range(RPT2 // ZROWS2):
            pltpu.sync_copy(
                zbuf, acc.at[pl.ds(s * RPT2 + kk * ZROWS2, ZROWS2)])
        plsc.subcore_barrier()

        wid = s * NC + c
        base_e = wid * EPT2

        @pl.loop(0, NCH2)
        def _(g):
            off = base_e + g * C2
            pltpu.sync_copy(src_hbm.at[pl.ds(off, C2)], src_v)
            pltpu.sync_copy(dst_hbm.at[pl.ds(off, C2)], dst_v)
            pltpu.sync_copy(sup_hbm.at[src_v], rows_v)
            pltpu.sync_copy(rows_v, acc.at[dst_v], add=True)

        plsc.subcore_barrier()
        pltpu.sync_copy(acc.at[pl.ds(s * RPT2, RPT2)],
                        out_hbm.at[pl.ds(c * N + s * RPT2, RPT2)])

    return k(sup2, src2, dst2)


def _final_body(p_ref, b2_ref, o_ref):
    a = p_ref[0] + p_ref[1] + b2_ref[...]
    m = jnp.max(a, axis=1, keepdims=True)
    ex = jnp.exp(a - m)
    lse = jnp.log(jnp.sum(ex, axis=1, keepdims=True))
    o_ref[...] = a - m - lse


def _final(parts, b2):
    return pl.pallas_call(
        _final_body,
        grid=(N // RB,),
        in_specs=[
            pl.BlockSpec((NC, RB, NCLASS), lambda r: (0, r, 0)),
            pl.BlockSpec((1, NCLASS), lambda r: (0, 0)),
        ],
        out_specs=pl.BlockSpec((RB, NCLASS), lambda r: (r, 0)),
        out_shape=jax.ShapeDtypeStruct((N, NCLASS), jnp.float32),
    )(parts, b2)


def kernel(x, adjs, W1, b1, W2, b2):
    sup1 = _supports(x, W1)

    step_off = (jnp.arange(NSTEP, dtype=jnp.int32) * N)[:, None]
    srcA = (adjs[:, 0, :] + step_off).reshape(-1)
    dstA = (adjs[:, 1, :] + step_off).reshape(-1)
    agg = _spmm1(sup1, srcA, dstA)

    h_pre = (agg.reshape(NC, NSTEP, N, HALF)
             .transpose(2, 1, 0, 3)
             .reshape(N, NSTEP * NHID))
    b1f = b1.reshape(1, NSTEP * NHID)
    sup2 = _layer2(h_pre, b1f, W2)

    parts = _spmm2(sup2, adjs[0, 0], adjs[0, 1])
    out = _final(parts.reshape(NC, N, NCLASS), b2.reshape(1, NCLASS))
    return out


# speedup vs baseline: 2.5506x; 2.5506x over previous
import functools

import jax
import jax.numpy as jnp
from jax import lax
from jax.experimental import pallas as pl
from jax.experimental.pallas import tpu as pltpu
from jax.experimental.pallas import tpu_sc as plsc

N = 10000
E = 320000
NSTEP = 3
NFEAT = 128
NHID = 128
NCLASS = 64

NC = 2
NS = 16
LANES = 16

NP = 10240
RPT = NP // NS
ZR = 64
ET0 = E // (NC * NS)
C = 128
ET = 10240
NCH = ET // C

RB = 1000


def _support_body(x_ref, w_ref, o_ref):
    o_ref[...] = jnp.dot(x_ref[...], w_ref[0],
                         preferred_element_type=jnp.float32)


def _supports(x, W1):
    return pl.pallas_call(
        _support_body,
        grid=(NSTEP, N // RB),
        in_specs=[
            pl.BlockSpec((RB, NFEAT), lambda s, r: (r, 0)),
            pl.BlockSpec((1, NFEAT, NHID), lambda s, r: (s, 0, 0)),
        ],
        out_specs=pl.BlockSpec(
            (RB, NHID), lambda s, r: (s * (N // RB) + r, 0)),
        out_shape=jax.ShapeDtypeStruct((NSTEP * N, NHID), jnp.float32),
    )(x, W1)


def _sc_spmm(nsteps, sup, srcp, dstp):
    mesh = plsc.VectorSubcoreMesh(core_axis_name="c", subcore_axis_name="s")

    @functools.partial(
        pl.kernel,
        mesh=mesh,
        out_type=jax.ShapeDtypeStruct((nsteps * NC * NP, NHID), jnp.float32),
        scratch_types=[
            pltpu.VMEM((C,), jnp.int32),
            pltpu.VMEM((C,), jnp.int32),
            pltpu.VMEM((C, NHID), jnp.float32),
            pltpu.VMEM((ZR, NHID), jnp.float32),
            pltpu.VMEM_SHARED((NP, NHID), jnp.float32),
        ],
    )
    def k(sup_hbm, src_hbm, dst_hbm, out_hbm, src_v, dst_v, rows_v, zbuf, acc):
        c = lax.axis_index("c")
        s = lax.axis_index("s")

        @pl.loop(0, ZR)
        def _(i):
            for j in range(NHID // LANES):
                zbuf[i, pl.ds(j * LANES, LANES)] = jnp.zeros(
                    (LANES,), jnp.float32)

        for st in range(nsteps):
            for kk in range(RPT // ZR):
                pltpu.sync_copy(zbuf, acc.at[pl.ds(s * RPT + kk * ZR, ZR)])
            plsc.subcore_barrier()

            base = ((st * NC + c) * NS + s) * ET

            @pl.loop(0, NCH)
            def _(g):
                off = base + g * C
                pltpu.sync_copy(src_hbm.at[pl.ds(off, C)], src_v)
                pltpu.sync_copy(dst_hbm.at[pl.ds(off, C)], dst_v)
                pltpu.sync_copy(sup_hbm.at[src_v], rows_v)
                pltpu.sync_copy(rows_v, acc.at[dst_v], add=True)

            plsc.subcore_barrier()
            pltpu.sync_copy(acc.at[pl.ds(s * RPT, RPT)],
                            out_hbm.at[pl.ds((st * NC + c) * NP + s * RPT,
                                             RPT)])

    return k(sup, srcp, dstp)


def _mid_body(p_ref, b1_ref, w_ref, o_ref):
    o = jnp.zeros((RB, NHID), jnp.float32)
    for s in range(NSTEP):
        h = jnp.maximum(p_ref[s, 0] + p_ref[s, 1] + b1_ref[s][None, :], 0.0)
        o = o + jnp.dot(h, w_ref[s], preferred_element_type=jnp.float32)
    o_ref[...] = o


def _mid(agg, b1, W2p):
    return pl.pallas_call(
        _mid_body,
        grid=(N // RB,),
        in_specs=[
            pl.BlockSpec((NSTEP, NC, RB, NHID), lambda r: (0, 0, r, 0)),
            pl.BlockSpec((NSTEP, NHID), lambda r: (0, 0)),
            pl.BlockSpec((NSTEP, NHID, NHID), lambda r: (0, 0, 0)),
        ],
        out_specs=pl.BlockSpec((RB, NHID), lambda r: (r, 0)),
        out_shape=jax.ShapeDtypeStruct((N, NHID), jnp.float32),
    )(agg, b1, W2p)


def _final_body(p_ref, b2_ref, o_ref):
    a = p_ref[0, :, :NCLASS] + p_ref[1, :, :NCLASS] + b2_ref[...]
    m = jnp.max(a, axis=1, keepdims=True)
    ex = jnp.exp(a - m)
    lse = jnp.log(jnp.sum(ex, axis=1, keepdims=True))
    o_ref[...] = a - m - lse


def _final(parts, b2):
    return pl.pallas_call(
        _final_body,
        grid=(N // RB,),
        in_specs=[
            pl.BlockSpec((NC, RB, NHID), lambda r: (0, r, 0)),
            pl.BlockSpec((1, NCLASS), lambda r: (0, 0)),
        ],
        out_specs=pl.BlockSpec((RB, NCLASS), lambda r: (r, 0)),
        out_shape=jax.ShapeDtypeStruct((N, NCLASS), jnp.float32),
    )(parts, b2)


def _pad_edges(src, dst):
    srcp = jnp.pad(src, ((0, 0), (0, 0), (0, ET - ET0)))
    dstp = jnp.pad(dst, ((0, 0), (0, 0), (0, ET - ET0)), constant_values=N)
    return srcp.reshape(-1), dstp.reshape(-1)


def kernel(x, adjs, W1, b1, W2, b2):
    sup1 = _supports(x, W1)

    step_off = (jnp.arange(NSTEP, dtype=jnp.int32) * N)[:, None]
    src1 = (adjs[:, 0, :] + step_off).reshape(NSTEP, NC * NS, ET0)
    dst1 = adjs[:, 1, :].reshape(NSTEP, NC * NS, ET0)
    src1, dst1 = _pad_edges(src1, dst1)
    agg = _sc_spmm(NSTEP, sup1, src1, dst1).reshape(NSTEP, NC, NP, NHID)

    W2p = jnp.pad(W2.reshape(NSTEP, NHID, NCLASS),
                  ((0, 0), (0, 0), (0, NHID - NCLASS)))
    sup2 = _mid(agg, b1, W2p)

    src2 = adjs[0, 0].reshape(1, NC * NS, ET0)
    dst2 = adjs[0, 1].reshape(1, NC * NS, ET0)
    src2, dst2 = _pad_edges(src2, dst2)
    parts = _sc_spmm(1, sup2, src2, dst2).reshape(NC, NP, NHID)

    return _final(parts, b2.reshape(1, NCLASS))


# speedup vs baseline: 3.3004x; 1.2940x over previous
import functools

import jax
import jax.numpy as jnp
from jax import lax
from jax.experimental import pallas as pl
from jax.experimental.pallas import tpu as pltpu
from jax.experimental.pallas import tpu_sc as plsc

N = 10000
E = 320000
NSTEP = 3
NFEAT = 128
NHID = 128
NCLASS = 64

NC = 2
NS = 16
LANES = 16

NP = 10240
RPT = NP // NS
ZR = 32
ET0 = E // (NC * NS)
C = 128
ET = 10240
NCH = ET // C

RB = 1000


def _support_body(x_ref, w_ref, o_ref):
    o_ref[...] = jnp.dot(x_ref[...], w_ref[0],
                         preferred_element_type=jnp.float32)


def _supports(x, W1):
    return pl.pallas_call(
        _support_body,
        grid=(NSTEP, N // RB),
        in_specs=[
            pl.BlockSpec((RB, NFEAT), lambda s, r: (r, 0)),
            pl.BlockSpec((1, NFEAT, NHID), lambda s, r: (s, 0, 0)),
        ],
        out_specs=pl.BlockSpec(
            (RB, NHID), lambda s, r: (s * (N // RB) + r, 0)),
        out_shape=jax.ShapeDtypeStruct((NSTEP * N, NHID), jnp.float32),
    )(x, W1)


NBUF = 2
EB = ET // 2
CHB = EB // C
NGB = CHB // NBUF


def _sc_spmm(nsteps, sup, srcp, dstp):
    mesh = plsc.VectorSubcoreMesh(core_axis_name="c", subcore_axis_name="s")

    @functools.partial(
        pl.kernel,
        mesh=mesh,
        out_type=jax.ShapeDtypeStruct((nsteps * NC * NP, NHID), jnp.float32),
        scratch_types=[
            pltpu.VMEM((EB,), jnp.int32),
            pltpu.VMEM((EB,), jnp.int32),
            pltpu.VMEM((NBUF, C, NHID), jnp.float32),
            pltpu.VMEM((ZR, NHID), jnp.float32),
            pltpu.VMEM_SHARED((NP, NHID), jnp.float32),
            pltpu.SemaphoreType.DMA,
            pltpu.SemaphoreType.DMA,
        ],
    )
    def k(sup_hbm, src_hbm, dst_hbm, out_hbm, src_v, dst_v, rows_v, zbuf,
          acc, sem0, sem1):
        c = lax.axis_index("c")
        s = lax.axis_index("s")
        sems = [sem0, sem1]

        @pl.loop(0, ZR)
        def _(i):
            for j in range(NHID // LANES):
                zbuf[i, pl.ds(j * LANES, LANES)] = jnp.zeros(
                    (LANES,), jnp.float32)

        for st in range(nsteps):
            for kk in range(RPT // ZR):
                pltpu.sync_copy(zbuf, acc.at[pl.ds(s * RPT + kk * ZR, ZR)])
            plsc.subcore_barrier()

            base = ((st * NC + c) * NS + s) * ET
            for blk in range(ET // EB):
                boff = base + blk * EB
                pltpu.sync_copy(src_hbm.at[pl.ds(boff, EB)], src_v)
                pltpu.sync_copy(dst_hbm.at[pl.ds(boff, EB)], dst_v)

                for b in range(NBUF):
                    pltpu.async_copy(
                        sup_hbm.at[src_v.at[pl.ds(b * C, C)]],
                        rows_v.at[b], sems[b])

                @pl.loop(0, NGB)
                def _(gg):
                    for b in range(NBUF):
                        g = gg * NBUF + b
                        pltpu.make_async_copy(
                            sup_hbm.at[pl.ds(0, C)], rows_v.at[b],
                            sems[b]).wait()
                        pltpu.sync_copy(
                            rows_v.at[b],
                            acc.at[dst_v.at[pl.ds(g * C, C)]], add=True)

                        @pl.when(g + NBUF < CHB)
                        def _():
                            pltpu.async_copy(
                                sup_hbm.at[
                                    src_v.at[pl.ds((g + NBUF) * C, C)]],
                                rows_v.at[b], sems[b])

            plsc.subcore_barrier()
            pltpu.sync_copy(acc.at[pl.ds(s * RPT, RPT)],
                            out_hbm.at[pl.ds((st * NC + c) * NP + s * RPT,
                                             RPT)])

    return k(sup, srcp, dstp)


def _mid_body(p_ref, b1_ref, w_ref, o_ref):
    o = jnp.zeros((RB, NHID), jnp.float32)
    for s in range(NSTEP):
        h = jnp.maximum(p_ref[s, 0] + p_ref[s, 1] + b1_ref[s][None, :], 0.0)
        o = o + jnp.dot(h, w_ref[s], preferred_element_type=jnp.float32)
    o_ref[...] = o


def _mid(agg, b1, W2p):
    return pl.pallas_call(
        _mid_body,
        grid=(N // RB,),
        in_specs=[
            pl.BlockSpec((NSTEP, NC, RB, NHID), lambda r: (0, 0, r, 0)),
            pl.BlockSpec((NSTEP, NHID), lambda r: (0, 0)),
            pl.BlockSpec((NSTEP, NHID, NHID), lambda r: (0, 0, 0)),
        ],
        out_specs=pl.BlockSpec((RB, NHID), lambda r: (r, 0)),
        out_shape=jax.ShapeDtypeStruct((N, NHID), jnp.float32),
    )(agg, b1, W2p)


def _final_body(p_ref, b2_ref, o_ref):
    a = p_ref[0, :, :NCLASS] + p_ref[1, :, :NCLASS] + b2_ref[...]
    m = jnp.max(a, axis=1, keepdims=True)
    ex = jnp.exp(a - m)
    lse = jnp.log(jnp.sum(ex, axis=1, keepdims=True))
    o_ref[...] = a - m - lse


def _final(parts, b2):
    return pl.pallas_call(
        _final_body,
        grid=(N // RB,),
        in_specs=[
            pl.BlockSpec((NC, RB, NHID), lambda r: (0, r, 0)),
            pl.BlockSpec((1, NCLASS), lambda r: (0, 0)),
        ],
        out_specs=pl.BlockSpec((RB, NCLASS), lambda r: (r, 0)),
        out_shape=jax.ShapeDtypeStruct((N, NCLASS), jnp.float32),
    )(parts, b2)


def _pad_edges(src, dst):
    srcp = jnp.pad(src, ((0, 0), (0, 0), (0, ET - ET0)))
    dstp = jnp.pad(dst, ((0, 0), (0, 0), (0, ET - ET0)), constant_values=N)
    return srcp.reshape(-1), dstp.reshape(-1)


def kernel(x, adjs, W1, b1, W2, b2):
    sup1 = _supports(x, W1)

    step_off = (jnp.arange(NSTEP, dtype=jnp.int32) * N)[:, None]
    src1 = (adjs[:, 0, :] + step_off).reshape(NSTEP, NC * NS, ET0)
    dst1 = adjs[:, 1, :].reshape(NSTEP, NC * NS, ET0)
    src1, dst1 = _pad_edges(src1, dst1)
    agg = _sc_spmm(NSTEP, sup1, src1, dst1).reshape(NSTEP, NC, NP, NHID)

    W2p = jnp.pad(W2.reshape(NSTEP, NHID, NCLASS),
                  ((0, 0), (0, 0), (0, NHID - NCLASS)))
    sup2 = _mid(agg, b1, W2p)

    src2 = adjs[0, 0].reshape(1, NC * NS, ET0)
    dst2 = adjs[0, 1].reshape(1, NC * NS, ET0)
    src2, dst2 = _pad_edges(src2, dst2)
    parts = _sc_spmm(1, sup2, src2, dst2).reshape(NC, NP, NHID)

    return _final(parts, b2.reshape(1, NCLASS))
